# baseline (speedup 1.0000x reference)
import functools

import jax
import jax.numpy as jnp
from jax import lax
from jax.experimental import pallas as pl
from jax.experimental.pallas import tpu as pltpu
from jax.experimental.pallas import tpu_sc as plsc

NC = 2
NS = 16
NW = NC * NS

SEQ = 200
EMBED = 64
LANES = 16
BBLK = 128
CH = 8
NCH = SEQ // CH
NBUF = 3


def _build(batch):
    assert batch == NW * BBLK
    ets = EMBED // 8
    mesh = plsc.VectorSubcoreMesh(core_axis_name="c", subcore_axis_name="s")

    @functools.partial(
        pl.kernel,
        mesh=mesh,
        compiler_params=pltpu.CompilerParams(
            use_tc_tiling_on_sc=False, needs_layout_passes=False),
        out_type=jax.ShapeDtypeStruct((SEQ, ets, NW, 8, BBLK), jnp.float32),
        scratch_types=[
            pltpu.VMEM((SEQ, EMBED), jnp.float32),
            pltpu.VMEM((2, CH, BBLK), jnp.int32),
            pltpu.VMEM((NBUF, BBLK, EMBED), jnp.float32),
            pltpu.VMEM((NBUF, ets, 1, 8, BBLK), jnp.float32),
            pltpu.SemaphoreType.DMA((NBUF,)),
            pltpu.SemaphoreType.DMA((NBUF,)),
        ],
    )
    def body(xt_hbm, tok_hbm, pos_hbm, out_hbm, pos_v, idx_v, rows_v, tile_v,
             gsems, osems):
        w = lax.axis_index("s") * NC + lax.axis_index("c")
        b0 = w * BBLK
        pltpu.sync_copy(pos_hbm, pos_v)

        def load_chunk(c):
            pltpu.sync_copy(xt_hbm.at[pl.ds(c * CH, CH), pl.ds(b0, BBLK)],
                            idx_v.at[lax.rem(c, 2)])

        def gather_refs(si, b):
            idxref = idx_v.at[lax.rem(si // CH, 2), lax.rem(si, CH)]
            return tok_hbm.at[idxref], rows_v.at[b]

        def issue_gather(si, b):
            src, dst = gather_refs(si, b)
            pltpu.async_copy(src, dst, gsems.at[b])

        def drain_gather(si, b):
            src, dst = gather_refs(si, b)
            pltpu.make_async_copy(src, dst, gsems.at[b]).wait()

        def out_refs(si, b):
            return tile_v.at[b], out_hbm.at[si, :, pl.ds(w, 1)]

        def issue_out(si, b):
            src, dst = out_refs(si, b)
            pltpu.async_copy(src, dst, osems.at[b])

        def drain_out(si, b):
            src, dst = out_refs(si, b)
            pltpu.make_async_copy(src, dst, osems.at[b]).wait()

        def transpose_add(si, b):
            rows = rows_v.at[b]
            tile = tile_v.at[b]
            iota = lax.broadcasted_iota(jnp.int32, (LANES,), 0)
            rowvs = [iota + bb * LANES for bb in range(BBLK // LANES)]
            siv = jnp.full((LANES,), si, dtype=jnp.int32)

            for e in range(EMBED):
                colv = jnp.full((LANES,), e, dtype=jnp.int32)
                pose = plsc.load_gather(pos_v, [siv, colv])
                for bb in range(BBLK // LANES):
                    val = plsc.load_gather(rows, [rowvs[bb], colv])
                    tile[e // 8, 0, e % 8, pl.ds(bb * LANES, LANES)] = val + pose

        load_chunk(0)
        issue_gather(0, 0)

        def step(si, carry):
            b = lax.rem(si, NBUF)

            @pl.when(jnp.logical_and(lax.rem(si, CH) == 0, si < (NCH - 1) * CH))
            def _():
                load_chunk(si // CH + 1)

            @pl.when(si >= NBUF)
            def _():
                drain_out(si - NBUF, b)

            @pl.when(si + 1 < SEQ)
            def _():
                issue_gather(si + 1, lax.rem(si + 1, NBUF))

            drain_gather(si, b)
            transpose_add(si, b)
            issue_out(si, b)
            return carry

        lax.fori_loop(0, SEQ, step, 0)

        for si in (SEQ - 3, SEQ - 2, SEQ - 1):
            drain_out(si, si % NBUF)

    return body


def kernel(x, token_table, pos_table):
    batch = x.shape[0]
    xt = jnp.transpose(x.astype(jnp.int32))
    tok_rm = jnp.reshape(token_table, (500000, 128))
    tok_rm = lax.optimization_barrier(tok_rm)
    tok2 = jnp.reshape(tok_rm, (1000000, 64))
    run = _build(batch)
    out5 = run(xt, tok2, pos_table)
    out = jnp.reshape(jnp.transpose(out5, (2, 4, 0, 1, 3)), (batch, SEQ, EMBED))
    return out


# speedup vs baseline: 1.9424x; 1.9424x over previous
import functools

import jax
import jax.numpy as jnp
from jax import lax
from jax.experimental import pallas as pl
from jax.experimental.pallas import tpu as pltpu
from jax.experimental.pallas import tpu_sc as plsc

NC = 2
NS = 16
NW = NC * NS

SEQ = 200
EMBED = 64
LANES = 16
VPR = EMBED // LANES

SPLITS = ((0, 104), (104, 96))

G = 2
NBUF = 3


def _build(batch):
    seqs_per_w = batch // NW
    ngroups = seqs_per_w // G
    mesh = plsc.VectorSubcoreMesh(core_axis_name="c", subcore_axis_name="s")

    @functools.partial(
        pl.kernel,
        mesh=mesh,
        compiler_params=pltpu.CompilerParams(
            use_tc_tiling_on_sc=False, needs_layout_passes=False),
        out_type=jax.ShapeDtypeStruct((batch, SEQ, 2 * EMBED), jnp.float32),
        scratch_types=[
            pltpu.VMEM((SEQ, EMBED), jnp.float32),
            pltpu.VMEM((NBUF, G, SEQ), jnp.int32),
            pltpu.VMEM((NBUF, G, SEQ, EMBED), jnp.float32),
            pltpu.SemaphoreType.DMA((NBUF,)),
            pltpu.SemaphoreType.DMA((NBUF,)),
        ],
    )
    def body(x_hbm, tok_hbm, pos_hbm, out_hbm, pos_v, idx_v, rows_v, gsems, osems):
        wid = lax.axis_index("s") * NC + lax.axis_index("c")
        base_seq = wid * seqs_per_w
        pltpu.sync_copy(pos_hbm, pos_v)

        def prefill(b):
            def row(r, c):
                for j in range(VPR):
                    sl = pl.ds(j * LANES, LANES)
                    v = pos_v[r, sl]
                    for s in range(G):
                        rows_v[b, s, r, sl] = v
                return c

            lax.fori_loop(0, SEQ, row, 0, unroll=2)

        def issue_gather(g, b):
            s0 = base_seq + g * G
            pltpu.sync_copy(x_hbm.at[pl.ds(s0, G)], idx_v.at[b])
            for s in range(G):
                for (o, n) in SPLITS:
                    pltpu.async_copy(
                        tok_hbm.at[idx_v.at[b, s, pl.ds(o, n)]],
                        rows_v.at[b, s, pl.ds(o, n)],
                        gsems.at[b], add=True)

        def drain_gather(b):
            for s in range(G):
                for (o, n) in SPLITS:
                    pltpu.make_async_copy(
                        tok_hbm.at[idx_v.at[b, s, pl.ds(o, n)]],
                        rows_v.at[b, s, pl.ds(o, n)],
                        gsems.at[b]).wait()

        def issue_out(g, b):
            s0 = base_seq + g * G
            pltpu.async_copy(rows_v.at[b],
                             out_hbm.at[pl.ds(s0, G), :, pl.ds(0, EMBED)],
                             osems.at[b])

        def drain_out(g, b):
            s0 = base_seq + g * G
            pltpu.make_async_copy(rows_v.at[b],
                                  out_hbm.at[pl.ds(s0, G), :, pl.ds(0, EMBED)],
                                  osems.at[b]).wait()

        prefill(0)
        issue_gather(0, 0)

        def step(g, carry):
            b = lax.rem(g, NBUF)
            bn = lax.rem(g + 1, NBUF)

            @pl.when(g >= 2)
            def _():
                drain_out(g - 2, bn)

            @pl.when(g + 1 < ngroups)
            def _():
                prefill(bn)
                issue_gather(g + 1, bn)

            drain_gather(b)
            issue_out(g, b)
            return carry

        lax.fori_loop(0, ngroups, step, 0)

        for g in (ngroups - 2, ngroups - 1):
            drain_out(g, g % NBUF)

    return body


def kernel(x, token_table, pos_table):
    batch = x.shape[0]
    tok_flat = lax.optimization_barrier(jnp.reshape(token_table, (-1,)))
    tok2 = jnp.reshape(tok_flat, token_table.shape)
    run = _build(batch)
    padded = run(x.astype(jnp.int32), tok2, pos_table)
    return padded[:, :, :EMBED]
